# baseline (speedup 1.0000x reference)
import jax
import jax.numpy as jnp
from jax.experimental import pallas as pl
from jax.experimental.pallas import tpu as pltpu

N = 4096
D_IN = 128
D_OUT = 128
NUM_BASES = 8
NUM_REL = 4

BLOCK_N = 1024
NUM_CORES = 2
A_CHUNKS = 4
CHUNK_K = N // A_CHUNKS


def _rgcn_kernel(*refs):
    a_refs = refs[:A_CHUNKS]
    x0_ref, x1_ref, x2_ref, x3_ref, wb_ref, wr_ref = refs[A_CHUNKS:A_CHUNKS + 6]
    out_ref, y_ref = refs[A_CHUNKS + 6:]

    @pl.when(pl.program_id(1) == 0)
    def _compute_y():
        wb = wb_ref[...]
        wr = wr_ref[...]
        w = jax.lax.dot_general(
            wr, wb.reshape(NUM_BASES, D_IN * D_OUT),
            (((1,), (0,)), ((), ())),
            preferred_element_type=jnp.float32,
        ).reshape(NUM_REL, D_IN, D_OUT)
        y = jnp.zeros((N, D_OUT), dtype=jnp.float32)
        for r, x_ref in enumerate((x0_ref, x1_ref, x2_ref, x3_ref)):
            y = y + jnp.dot(x_ref[...], w[r].astype(jnp.bfloat16),
                            preferred_element_type=jnp.float32)
        y_ref[...] = y.astype(jnp.bfloat16)

    acc = jnp.zeros((BLOCK_N, D_OUT), dtype=jnp.float32)
    for c in range(A_CHUNKS):
        acc = acc + jnp.dot(a_refs[c][...].astype(jnp.bfloat16),
                            y_ref[c * CHUNK_K:(c + 1) * CHUNK_K, :],
                            preferred_element_type=jnp.float32)
    out_ref[...] = acc


def kernel(a, x, w_bases, w_rel):
    xs = [x[:, :, r].astype(jnp.bfloat16) for r in range(NUM_REL)]
    inner = N // (NUM_CORES * BLOCK_N)
    x_spec = pl.BlockSpec((N, D_IN), lambda i, j: (0, 0))

    def a_spec(c):
        return pl.BlockSpec((BLOCK_N, CHUNK_K),
                            lambda i, j, c=c: (i * inner + j, c))

    return pl.pallas_call(
        _rgcn_kernel,
        grid=(NUM_CORES, inner),
        in_specs=[a_spec(c) for c in range(A_CHUNKS)] + [
            x_spec, x_spec, x_spec, x_spec,
            pl.BlockSpec((NUM_BASES, D_IN, D_OUT), lambda i, j: (0, 0, 0)),
            pl.BlockSpec((NUM_REL, NUM_BASES), lambda i, j: (0, 0)),
        ],
        out_specs=pl.BlockSpec((BLOCK_N, D_OUT), lambda i, j: (i * inner + j, 0)),
        out_shape=jax.ShapeDtypeStruct((N, D_OUT), jnp.float32),
        scratch_shapes=[pltpu.VMEM((N, D_OUT), jnp.bfloat16)],
        compiler_params=pltpu.CompilerParams(
            dimension_semantics=("parallel", "arbitrary"),
        ),
    )(*([a] * A_CHUNKS), *xs, w_bases, w_rel)


# speedup vs baseline: 1.0439x; 1.0439x over previous
import jax
import jax.numpy as jnp
from jax.experimental import pallas as pl
from jax.experimental.pallas import tpu as pltpu

N = 4096
D_IN = 128
D_OUT = 128
NUM_BASES = 8
NUM_REL = 4

BLOCK_N = 1024
NUM_CORES = 2
A_CHUNKS = 1
CHUNK_K = N // A_CHUNKS


def _rgcn_kernel(*refs):
    a_refs = refs[:A_CHUNKS]
    x0_ref, x1_ref, x2_ref, x3_ref, wb_ref, wr_ref = refs[A_CHUNKS:A_CHUNKS + 6]
    out_ref, y_ref = refs[A_CHUNKS + 6:]

    @pl.when(pl.program_id(1) == 0)
    def _compute_y():
        wb = wb_ref[...]
        wr = wr_ref[...]
        w = jax.lax.dot_general(
            wr, wb.reshape(NUM_BASES, D_IN * D_OUT),
            (((1,), (0,)), ((), ())),
            preferred_element_type=jnp.float32,
        ).reshape(NUM_REL, D_IN, D_OUT)
        y = jnp.zeros((N, D_OUT), dtype=jnp.float32)
        for r, x_ref in enumerate((x0_ref, x1_ref, x2_ref, x3_ref)):
            y = y + jnp.dot(x_ref[...], w[r].astype(jnp.bfloat16),
                            preferred_element_type=jnp.float32)
        y_ref[...] = y.astype(jnp.bfloat16)

    acc = jnp.zeros((BLOCK_N, D_OUT), dtype=jnp.float32)
    for c in range(A_CHUNKS):
        acc = acc + jnp.dot(a_refs[c][...].astype(jnp.bfloat16),
                            y_ref[c * CHUNK_K:(c + 1) * CHUNK_K, :],
                            preferred_element_type=jnp.float32)
    out_ref[...] = acc


def kernel(a, x, w_bases, w_rel):
    xs = [x[:, :, r].astype(jnp.bfloat16) for r in range(NUM_REL)]
    inner = N // (NUM_CORES * BLOCK_N)
    x_spec = pl.BlockSpec((N, D_IN), lambda i, j: (0, 0))

    def a_spec(c):
        return pl.BlockSpec((BLOCK_N, CHUNK_K),
                            lambda i, j, c=c: (i * inner + j, c))

    return pl.pallas_call(
        _rgcn_kernel,
        grid=(NUM_CORES, inner),
        in_specs=[a_spec(c) for c in range(A_CHUNKS)] + [
            x_spec, x_spec, x_spec, x_spec,
            pl.BlockSpec((NUM_BASES, D_IN, D_OUT), lambda i, j: (0, 0, 0)),
            pl.BlockSpec((NUM_REL, NUM_BASES), lambda i, j: (0, 0)),
        ],
        out_specs=pl.BlockSpec((BLOCK_N, D_OUT), lambda i, j: (i * inner + j, 0)),
        out_shape=jax.ShapeDtypeStruct((N, D_OUT), jnp.float32),
        scratch_shapes=[pltpu.VMEM((N, D_OUT), jnp.bfloat16)],
        compiler_params=pltpu.CompilerParams(
            dimension_semantics=("parallel", "arbitrary"),
        ),
    )(*([a] * A_CHUNKS), *xs, w_bases, w_rel)
